# baseline (speedup 1.0000x reference)
import functools

import jax
import jax.numpy as jnp
from jax import lax
from jax.experimental import pallas as pl
from jax.experimental.pallas import tpu as pltpu
from jax.experimental.pallas import tpu_sc as plsc

_NC, _NS, _L = 2, 16, 16
_NT = _NC * _NS
_B, _F, _D = 4096, 26, 64
_W = 26
_WI = 128
_RPW = _W * _WI


def _body(ids_hbm, offs_hbm, tab_hbm, out_hbm,
          idx_v, offs_v, buf0, buf1, sem0, sem1):
    c = lax.axis_index("c")
    s = lax.axis_index("s")
    w = c * _NS + s
    base = w * _RPW
    pltpu.sync_copy(ids_hbm.at[w], idx_v)
    pltpu.sync_copy(offs_hbm, offs_v)
    for g in range(_W):
        for q in range(_WI // _L):
            sl = pl.ds(q * _L, _L)
            idx_v[g, sl] = idx_v[g, sl] + offs_v[g, sl]

    bufs = (buf0, buf1)
    sems = (sem0, sem1)
    pltpu.async_copy(tab_hbm.at[idx_v.at[0]], buf0, sem0)
    pltpu.async_copy(tab_hbm.at[idx_v.at[1]], buf1, sem1)

    def outer(i, carry):
        for b in range(2):
            g = i * 2 + b
            pltpu.make_async_copy(
                tab_hbm.at[idx_v.at[g]], bufs[b], sems[b]).wait()
            pltpu.sync_copy(bufs[b],
                            out_hbm.at[pl.ds(base + g * _WI, _WI)])

            @pl.when(g + 2 < _W)
            def _():
                pltpu.async_copy(
                    tab_hbm.at[idx_v.at[g + 2]], bufs[b], sems[b])
        return carry

    lax.fori_loop(0, _W // 2, outer, 0)


@jax.jit
def _sc_gather(ids_t, offs_rep, table):
    mesh = plsc.VectorSubcoreMesh(core_axis_name="c", subcore_axis_name="s")
    f = pl.kernel(
        _body,
        out_type=jax.ShapeDtypeStruct((_B * _F, _D), jnp.float32),
        mesh=mesh,
        scratch_types=[
            pltpu.VMEM((_W, _WI), jnp.int32),
            pltpu.VMEM((_W, _WI), jnp.int32),
            pltpu.VMEM((_WI, _D), jnp.float32),
            pltpu.VMEM((_WI, _D), jnp.float32),
            pltpu.SemaphoreType.DMA,
            pltpu.SemaphoreType.DMA,
        ],
        compiler_params=pltpu.CompilerParams(
            needs_layout_passes=False, use_tc_tiling_on_sc=False),
    )
    return f(ids_t, offs_rep, table)


def kernel(hash_ids, table, offsets):
    ids_t = hash_ids.astype(jnp.int32).reshape(_NT, _W, _WI)
    offs_rep = jnp.tile(offsets.astype(jnp.int32), _WI).reshape(_W, _WI)
    out = _sc_gather(ids_t, offs_rep, table)
    return out.reshape(_B, _F, _D)


# speedup vs baseline: 4.9963x; 4.9963x over previous
import functools

import jax
import jax.numpy as jnp
from jax import lax
from jax.experimental import pallas as pl
from jax.experimental.pallas import tpu as pltpu
from jax.experimental.pallas import tpu_sc as plsc

_NC, _NS, _L = 2, 16, 16
_NT = _NC * _NS
_B, _F, _D = 4096, 26, 64
_RB = 100096
_HALF = _RB // 2
_NU = _F * 2


def _body(ids_hbm, offs_hbm, tab_hbm, out_hbm,
          offs_v, idc_v, rel0_v, rel1_v, val_v, bufA, bufB, semA, semB):
    c = lax.axis_index("c")
    s = lax.axis_index("s")
    t = c * _NS + s
    pltpu.sync_copy(offs_hbm, offs_v)
    bufA[pl.ds(_HALF, _L)] = lax.full((_L,), 0.0, jnp.float32)
    bufB[pl.ds(_HALF, _L)] = lax.full((_L,), 0.0, jnp.float32)

    def off_at(f):
        return offs_v[0, pl.ds(f, _L)][0]

    def chunk_src(k):
        f = lax.div(k, 4)
        r = lax.rem(k, 4)
        j = lax.div(r, 2)
        h = lax.rem(r, 2)
        off = off_at(f)
        rb = pl.multiple_of(lax.bitwise_and(off, ~127), 128)
        return tab_hbm.at[t + j * _NT, pl.ds(rb + h * _HALF, _HALF)]

    pltpu.async_copy(chunk_src(0), bufA.at[pl.ds(0, _HALF)], semA)
    pltpu.async_copy(chunk_src(1), bufB.at[pl.ds(0, _HALF)], semB)

    def pair_step(k2, carry):
        f = lax.div(k2, 2)
        j = lax.rem(k2, 2)

        @pl.when(j == 0)
        def _():
            pltpu.sync_copy(ids_hbm.at[f], idc_v)
            off = off_at(f)
            base = off - lax.bitwise_and(off, ~127)
            sent = lax.full((_L,), _HALF, jnp.int32)
            lim = lax.full((_L,), _HALF, jnp.uint32)
            for g in range(_B // _L):
                sl = pl.ds(g * _L, _L)
                rel = idc_v[0, sl] + base
                in0 = lax.lt(plsc.bitcast(rel, jnp.uint32), lim)
                rel0_v[0, sl] = lax.select(in0, rel, sent)
                relm = rel - _HALF
                in1 = lax.lt(plsc.bitcast(relm, jnp.uint32), lim)
                rel1_v[0, sl] = lax.select(in1, relm, sent)

        pltpu.make_async_copy(chunk_src(2 * k2),
                              bufA.at[pl.ds(0, _HALF)], semA).wait()
        for g in range(_B // _L):
            sl = pl.ds(g * _L, _L)
            val_v[0, sl] = plsc.load_gather(bufA, [rel0_v[0, sl]])

        @pl.when(k2 < _NU - 1)
        def _():
            pltpu.async_copy(chunk_src(2 * k2 + 2),
                             bufA.at[pl.ds(0, _HALF)], semA)

        pltpu.make_async_copy(chunk_src(2 * k2 + 1),
                              bufB.at[pl.ds(0, _HALF)], semB).wait()
        for g in range(_B // _L):
            sl = pl.ds(g * _L, _L)
            val_v[0, sl] = val_v[0, sl] + plsc.load_gather(
                bufB, [rel1_v[0, sl]])

        @pl.when(k2 < _NU - 1)
        def _():
            pltpu.async_copy(chunk_src(2 * k2 + 3),
                             bufB.at[pl.ds(0, _HALF)], semB)

        pltpu.sync_copy(val_v.at[0], out_hbm.at[f, t + j * _NT])
        return carry

    lax.fori_loop(0, _NU, pair_step, 0)


@jax.jit
def _sc_gather(ids_t, offs, tab_t):
    mesh = plsc.VectorSubcoreMesh(core_axis_name="c", subcore_axis_name="s")
    f = pl.kernel(
        _body,
        out_type=jax.ShapeDtypeStruct((_F, _D, _B), jnp.float32),
        mesh=mesh,
        scratch_types=[
            pltpu.VMEM((1, 48), jnp.int32),
            pltpu.VMEM((1, _B), jnp.int32),
            pltpu.VMEM((1, _B), jnp.int32),
            pltpu.VMEM((1, _B), jnp.int32),
            pltpu.VMEM((1, _B), jnp.float32),
            pltpu.VMEM((_HALF + _L,), jnp.float32),
            pltpu.VMEM((_HALF + _L,), jnp.float32),
            pltpu.SemaphoreType.DMA,
            pltpu.SemaphoreType.DMA,
        ],
        compiler_params=pltpu.CompilerParams(needs_layout_passes=False),
    )
    return f(ids_t, offs, tab_t)


def kernel(hash_ids, table, offsets):
    ids_t = hash_ids.astype(jnp.int32).T.reshape(_F, 1, _B)
    offs = jnp.zeros((1, 48), jnp.int32).at[0, :_F].set(
        offsets.astype(jnp.int32))
    out = _sc_gather(ids_t, offs, table.T)
    return out.transpose(2, 0, 1)
